# baseline (speedup 1.0000x reference)
import functools
import math

import jax
import jax.numpy as jnp
from jax import lax
from jax.experimental import pallas as pl
from jax.experimental.pallas import tpu as pltpu
from jax.experimental.pallas import tpu_sc as plsc

D_MODEL = 64
SCALE = math.sqrt(D_MODEL)

_NC = 2
_NS = 16
_NW = _NC * _NS
_NBUF = 4

_V = 1000000
_NT_FULL = _V // 128
_T_PER_W = (-(-_NT_FULL // _NW) + 1) // 2 * 2


@jax.jit
def _sc_transpose_table(table_t, tail):
    mesh = plsc.VectorSubcoreMesh(core_axis_name="c", subcore_axis_name="s")

    @functools.partial(
        pl.kernel,
        mesh=mesh,
        out_type=jax.ShapeDtypeStruct((_V // 2, 128), jnp.float32),
        scratch_types=[
            pltpu.VMEM((64, 128), jnp.float32),
            pltpu.VMEM((64, 128), jnp.float32),
            pltpu.VMEM((64, 128), jnp.float32),
            pltpu.VMEM((64, 128), jnp.float32),
            pltpu.SemaphoreType.DMA,
            pltpu.SemaphoreType.DMA,
            pltpu.SemaphoreType.DMA,
            pltpu.SemaphoreType.DMA,
        ],
        compiler_params=pltpu.CompilerParams(use_tc_tiling_on_sc=True, needs_layout_passes=False),
    )
    def k1(tt_hbm, tail_hbm, z_hbm, in0, in1, ou0, ou1, gi0, gi1, go0, go1):
        inb, oub = (in0, in1), (ou0, ou1)
        sem_i, sem_o = (gi0, gi1), (go0, go1)
        wid = lax.axis_index("s") * _NC + lax.axis_index("c")
        t_base = wid * _T_PER_W

        def tile_of(k):
            return jnp.minimum(t_base + k, _NT_FULL - 1)

        def start_read(k, b):
            t = tile_of(k)
            return pltpu.async_copy(
                tt_hbm.at[:, pl.ds(t * 128, 128)], inb[b], sem_i[b])

        def wait_read(b):
            pltpu.make_async_copy(
                tt_hbm.at[:, pl.ds(0, 128)], inb[b], sem_i[b]).wait()

        def start_write(k, b):
            t = tile_of(k)
            return pltpu.async_copy(
                oub[b], z_hbm.at[pl.ds(t * 64, 64), :], sem_o[b])

        def wait_write(b):
            pltpu.make_async_copy(
                oub[b], z_hbm.at[pl.ds(0, 64), :], sem_o[b]).wait()

        iota = jax.lax.iota(jnp.int32, 16)
        rots = [(iota + i) & 15 for i in range(16)]

        def transpose(b):
            def rblk(rb, c2):
                r0 = rb * 16
                rvecs = [r0 + rots[i] for i in range(16)]
                for half in range(2):
                    icols = [2 * rv + half for rv in rvecs]
                    for cb in range(4):
                        c0 = cb * 16
                        crow = c0 + iota
                        ccol = c0 + half * 64 + iota
                        for i in range(16):
                            v = plsc.load_gather(inb[b], [crow, icols[i]])
                            plsc.store_scatter(oub[b], [rvecs[i], ccol], v)
                return c2
            lax.fori_loop(0, 4, rblk, 0, unroll=2)

        start_read(0, 0)
        start_read(1, 1)
        wait_read(0)
        transpose(0)
        start_write(0, 0)
        start_read(2, 0)
        wait_read(1)
        transpose(1)
        start_write(1, 1)

        def steady(p, carry):
            for t in range(2):
                k = 2 + p * 2 + t
                b = t
                start_read(k + 1, 1 - t)
                wait_read(b)
                wait_write(b)
                transpose(b)
                start_write(k, b)
            return carry

        lax.fori_loop(0, (_T_PER_W - 4) // 2, steady, 0)

        for k in (_T_PER_W - 2, _T_PER_W - 1):
            b = k % 2
            if k + 1 < _T_PER_W:
                start_read(k + 1, (k + 1) % 2)
            wait_read(b)
            wait_write(b)
            transpose(b)
            start_write(k, b)
        for b in range(2):
            wait_write(b)

        @pl.when(wid == _NW - 1)
        def _():
            pltpu.sync_copy(tail_hbm, z_hbm.at[pl.ds(_V // 2 - 32, 32), :])

    return k1(table_t, tail)


@functools.partial(jax.jit, static_argnames=("nrows", "seq"))
def _sc_embed(table, x, *, nrows, seq):
    rows_per_w = nrows // _NW
    mesh = plsc.VectorSubcoreMesh(core_axis_name="c", subcore_axis_name="s")

    @functools.partial(
        pl.kernel,
        mesh=mesh,
        out_type=jax.ShapeDtypeStruct((nrows, seq, D_MODEL), jnp.float32),
        scratch_types=[
            pltpu.VMEM((rows_per_w, seq), jnp.int32),
        ]
        + [pltpu.VMEM((seq, D_MODEL), jnp.float32) for _ in range(_NBUF)]
        + [pltpu.SemaphoreType.DMA for _ in range(2 * _NBUF)],
        compiler_params=pltpu.CompilerParams(use_tc_tiling_on_sc=False),
    )
    def k(table_hbm, x_hbm, out_hbm, idx_v, *bufs_and_sems):
        bufs = bufs_and_sems[:_NBUF]
        sem_g = bufs_and_sems[_NBUF:2 * _NBUF]
        sem_s = bufs_and_sems[2 * _NBUF:]

        wid = lax.axis_index("s") * _NC + lax.axis_index("c")
        row0 = wid * rows_per_w
        pltpu.sync_copy(x_hbm.at[pl.ds(row0, rows_per_w), :], idx_v)

        def start_gather(g, b):
            return pltpu.async_copy(table_hbm.at[idx_v.at[g]], bufs[b], sem_g[b])

        def wait_gather(g, b):
            pltpu.make_async_copy(
                table_hbm.at[idx_v.at[g]], bufs[b], sem_g[b]).wait()

        def start_scatter(g, b):
            return pltpu.async_copy(bufs[b], out_hbm.at[row0 + g], sem_s[b])

        def wait_scatter(b):
            pltpu.make_async_copy(bufs[b], out_hbm.at[row0], sem_s[b]).wait()

        def scale(b):
            def row_body(i, c2):
                for j in range(D_MODEL // 16):
                    sl = pl.ds(j * 16, 16)
                    bufs[b][i, sl] = bufs[b][i, sl] * SCALE
                return c2
            lax.fori_loop(0, seq, row_body, 0, unroll=4)

        n = rows_per_w
        start_gather(0, 0)
        start_gather(1, 1)
        start_gather(2, 2)
        wait_gather(0, 0)
        scale(0)
        start_scatter(0, 0)
        start_gather(3, 3)
        wait_gather(1, 1)
        scale(1)
        start_scatter(1, 1)

        def steady(p, carry):
            for b in range(_NBUF):
                g = 2 + p * _NBUF + b
                bb = (2 + b) % _NBUF
                bn = b % _NBUF
                wait_scatter(bn)
                start_gather(g + 2, bn)
                wait_gather(g, bb)
                scale(bb)
                start_scatter(g, bb)
            return carry

        lax.fori_loop(0, (n - 4) // _NBUF, steady, 0)

        wait_gather(n - 2, (n - 2) % _NBUF)
        scale((n - 2) % _NBUF)
        start_scatter(n - 2, (n - 2) % _NBUF)
        wait_gather(n - 1, (n - 1) % _NBUF)
        scale((n - 1) % _NBUF)
        start_scatter(n - 1, (n - 1) % _NBUF)
        for b in range(_NBUF):
            wait_scatter(b)

    return k(table, x)


def kernel(x, table):
    if x.dtype != jnp.int32:
        x = x.astype(jnp.int32)
    nrows, seq = x.shape
    table_t = table.T
    tail = lax.slice(table, (_V - 64, 0), (_V, D_MODEL)).reshape(32, 128)
    z = _sc_transpose_table(table_t, tail)
    z1 = z.reshape(_V, D_MODEL)
    return _sc_embed(z1, x, nrows=nrows, seq=seq)
